# baseline (speedup 1.0000x reference)
import functools

import jax
import jax.numpy as jnp
from jax import lax
from jax.experimental import pallas as pl
from jax.experimental.pallas import tpu as pltpu
from jax.experimental.pallas import tpu_sc as plsc

_NC = 2
_NS = 16
_NW = _NC * _NS
_CHUNK = 128
_REC = 3 * _CHUNK
_BN = 2000


def _sc_mesh():
    return plsc.VectorSubcoreMesh(core_axis_name="c", subcore_axis_name="s")


def _make_deg_kernel(n, e_pad):
    cpw = e_pad // (_NW * _CHUNK)

    @functools.partial(
        pl.kernel,
        mesh=_sc_mesh(),
        out_type=jax.ShapeDtypeStruct((_NC * n,), jnp.float32),
        scratch_types=[
            pltpu.VMEM((cpw * _REC,), jnp.int32),
            pltpu.VMEM((_CHUNK,), jnp.int32),
            pltpu.VMEM((_CHUNK,), jnp.float32),
            pltpu.VMEM((n,), jnp.float32),
            pltpu.VMEM_SHARED((n,), jnp.float32),
        ],
    )
    def deg_kernel(pk_hbm, out_hbm, pk_all, dst_v, w_v, zero_v, acc):
        c = lax.axis_index("c")
        s = lax.axis_index("s")
        wid = c * _NS + s
        base = pl.multiple_of(wid * cpw * _REC, 8)

        pltpu.sync_copy(pk_hbm.at[pl.ds(base, cpw * _REC)], pk_all)

        @pl.when(s == 0)
        def _zero():
            zeros = jnp.zeros((16,), jnp.float32)

            def zb(i, carry):
                zero_v[pl.ds(pl.multiple_of(i * 16, 16), 16)] = zeros
                return carry

            lax.fori_loop(0, n // 16, zb, 0)
            pltpu.sync_copy(zero_v, acc)

        plsc.subcore_barrier()

        def body(k, carry):
            def grp(gi, carry2):
                off = pl.multiple_of(k * _REC + gi * 16, 16)
                dst_v[pl.ds(gi * 16, 16)] = pk_all[pl.ds(off + _CHUNK, 16)]
                w_v[pl.ds(gi * 16, 16)] = jnp.exp(
                    lax.bitcast_convert_type(
                        pk_all[pl.ds(off + 2 * _CHUNK, 16)], jnp.float32))
                return carry2

            lax.fori_loop(0, _CHUNK // 16, grp, 0)
            pltpu.sync_copy(w_v, acc.at[dst_v], add=True)
            return carry

        lax.fori_loop(0, cpw, body, 0)
        plsc.subcore_barrier()

        q = n // 5
        @pl.when(s < 5)
        def _write():
            pltpu.sync_copy(acc.at[pl.ds(pl.multiple_of(s * q, 8), q)],
                            zero_v.at[pl.ds(0, q)])
            pltpu.sync_copy(
                zero_v.at[pl.ds(0, q)],
                out_hbm.at[pl.ds(pl.multiple_of(c * n + s * q, 8), q)],
            )

    return deg_kernel


def _make_agg_kernel(n, cpw0, cpw1, d):
    nt0 = cpw0 // 3
    nt1 = cpw1 // 3
    slab = ((-(-n // _NS) + 7) // 8) * 8
    last_slab = n - (_NS - 1) * slab

    @functools.partial(
        pl.kernel,
        mesh=_sc_mesh(),
        out_type=jax.ShapeDtypeStruct((_NC * n, d), jnp.float32),
        scratch_types=[
            pltpu.VMEM((_CHUNK, d), jnp.float32),
            pltpu.VMEM((_CHUNK, d), jnp.float32),
            pltpu.VMEM((_CHUNK, d), jnp.float32),
            pltpu.VMEM((_REC,), jnp.int32),
            pltpu.VMEM((_REC,), jnp.int32),
            pltpu.VMEM((_REC,), jnp.int32),
            pltpu.VMEM((_CHUNK,), jnp.int32),
            pltpu.VMEM((_CHUNK,), jnp.int32),
            pltpu.VMEM((_CHUNK,), jnp.int32),
            pltpu.VMEM_SHARED((n, d), jnp.float32),
            pltpu.SemaphoreType.DMA,
            pltpu.SemaphoreType.DMA,
            pltpu.SemaphoreType.DMA,
            pltpu.SemaphoreType.DMA,
            pltpu.SemaphoreType.DMA,
            pltpu.SemaphoreType.DMA,
            pltpu.SemaphoreType.DMA,
            pltpu.SemaphoreType.DMA,
            pltpu.SemaphoreType.DMA,
        ],
    )
    def agg_kernel(g_hbm, pk_hbm, out_hbm,
                   r0, r1, r2, eb0, eb1, eb2, db0, db1, db2, acc,
                   es0, es1, es2, gs0, gs1, gs2, ss0, ss1, ss2):
        c_ax = lax.axis_index("c")
        s_ax = lax.axis_index("s")
        cbase = jnp.where(c_ax == 0, s_ax * cpw0,
                          _NS * cpw0 + s_ax * cpw1)
        nt_dyn = jnp.where(c_ax == 0, nt0, nt1)

        rows = (r0, r1, r2)
        ebs = (eb0, eb1, eb2)
        dbs = (db0, db1, db2)
        esem = (es0, es1, es2)
        gsem = (gs0, gs1, gs2)
        ssem = (ss0, ss1, ss2)

        zeros = jnp.zeros((16,), jnp.float32)

        def zb(i, carry):
            for j in range(d // 16):
                r0[i, pl.ds(j * 16, 16)] = zeros
            return carry

        lax.fori_loop(0, _CHUNK, zb, 0)

        row0 = s_ax * slab

        def zero_slab(nrows):
            full = nrows // _CHUNK
            rem = nrows - full * _CHUNK
            for i in range(full):
                pltpu.sync_copy(
                    r0,
                    acc.at[pl.ds(pl.multiple_of(row0 + i * _CHUNK, 8),
                                 _CHUNK)])
            if rem:
                pltpu.sync_copy(
                    r0.at[pl.ds(0, rem)],
                    acc.at[pl.ds(pl.multiple_of(row0 + full * _CHUNK, 8),
                                 rem)])

        @pl.when(s_ax < _NS - 1)
        def _z0():
            zero_slab(slab)

        @pl.when(s_ax == _NS - 1)
        def _z1():
            zero_slab(last_slab)

        plsc.subcore_barrier()

        def start_eload(k, u):
            gid = pl.multiple_of((cbase + k) * _REC, 8)
            pltpu.async_copy(pk_hbm.at[pl.ds(gid, _REC)], ebs[u], esem[u])

        def wait_eload(u):
            pltpu.make_async_copy(pk_hbm.at[pl.ds(0, _REC)], ebs[u],
                                  esem[u]).wait()

        def start_gather(u):
            idx = ebs[u].at[pl.ds(0, _CHUNK)]
            pltpu.async_copy(g_hbm.at[idx], rows[u], gsem[u])

        def wait_gather(u):
            pltpu.make_async_copy(g_hbm.at[pl.ds(0, _CHUNK)], rows[u],
                                  gsem[u]).wait()

        def start_scatter(u):
            pltpu.async_copy(rows[u], acc.at[dbs[u]], ssem[u], add=True)

        def wait_scatter(u):
            pltpu.make_async_copy(g_hbm.at[pl.ds(0, _CHUNK)], rows[u],
                                  ssem[u]).wait()

        def process(u):
            ru = rows[u]
            eu = ebs[u]
            du = dbs[u]

            def cp(gi, carry):
                off = pl.multiple_of(gi * 16, 16)
                du[pl.ds(off, 16)] = eu[pl.ds(off + _CHUNK, 16)]
                return carry

            lax.fori_loop(0, _CHUNK // 16, cp, 0)

            orv = eu[pl.ds(2 * _CHUNK, 16)]
            for gi in range(1, _CHUNK // 16):
                orv = orv | eu[pl.ds(2 * _CHUNK + gi * 16, 16)]

            tot = orv[0]
            for l in range(1, 16):
                tot = tot | orv[l]

            @pl.when(tot != 0)
            def _scale():
                def grp(gi, carry):
                    off = pl.multiple_of(gi * 16, 16)
                    w16 = jnp.exp(lax.bitcast_convert_type(
                        eu[pl.ds(off + 2 * _CHUNK, 16)], jnp.float32))
                    for l in range(16):
                        we = w16[l]
                        e = gi * 16 + l
                        for j in range(d // 16):
                            ru[e, pl.ds(j * 16, 16)] = (
                                ru[e, pl.ds(j * 16, 16)] * we)
                    return carry

                lax.fori_loop(0, _CHUNK // 16, grp, 0)

        start_eload(0, 0)
        start_eload(1, 1)
        wait_eload(0)
        start_gather(0)

        def body(t, carry):
            for u in range(3):
                c = t * 3 + u
                nxt = (u + 2) % 3
                prv = (u + 1) % 3
                if u == 0:
                    start_eload(c + 2, nxt)
                    wait_eload(prv)

                    @pl.when(t > 0)
                    def _ws():
                        wait_scatter(prv)
                    start_gather(prv)
                elif u == 1:
                    @pl.when(t < nt_dyn - 1)
                    def _el():
                        start_eload(c + 2, nxt)
                    wait_eload(prv)

                    @pl.when(t > 0)
                    def _ws1():
                        wait_scatter(prv)
                    start_gather(prv)
                else:
                    @pl.when(t < nt_dyn - 1)
                    def _el2():
                        start_eload(c + 2, nxt)
                    wait_scatter(prv)

                    @pl.when(t < nt_dyn - 1)
                    def _g2():
                        wait_eload(prv)
                        start_gather(prv)
                wait_gather(u)
                process(u)
                start_scatter(u)
            return carry

        lax.fori_loop(0, nt_dyn, body, 0)
        wait_scatter(1)
        wait_scatter(2)
        plsc.subcore_barrier()

        def writeout(nrows):
            pltpu.sync_copy(
                acc.at[pl.ds(pl.multiple_of(row0, 8), nrows)],
                out_hbm.at[pl.ds(pl.multiple_of(c_ax * n + row0, 8), nrows)],
            )

        @pl.when(s_ax < _NS - 1)
        def _w0():
            writeout(slab)

        @pl.when(s_ax == _NS - 1)
        def _w1():
            writeout(last_slab)

    return agg_kernel


def _tc_mm(x, w):
    n, k = x.shape
    m = w.shape[1]

    def body(x_ref, w_ref, o_ref):
        o_ref[...] = jnp.dot(x_ref[...], w_ref[...],
                             preferred_element_type=jnp.float32)

    return pl.pallas_call(
        body,
        grid=(n // _BN,),
        in_specs=[pl.BlockSpec((_BN, k), lambda i: (i, 0)),
                  pl.BlockSpec((k, m), lambda i: (0, 0))],
        out_specs=pl.BlockSpec((_BN, m), lambda i: (i, 0)),
        out_shape=jax.ShapeDtypeStruct((n, m), jnp.float32),
    )(x, w)


def _tc_finalize(d0, d1, hw):
    n, dh = hw.shape

    def body(d0_ref, d1_ref, hw_ref, g_ref, dinv_ref):
        deg = d0_ref[...] + d1_ref[...] + 1.0
        dinv = lax.rsqrt(deg)
        dinv_ref[...] = dinv
        g_ref[...] = dinv * hw_ref[...]

    return pl.pallas_call(
        body,
        grid=(n // _BN,),
        in_specs=[pl.BlockSpec((_BN, 1), lambda i: (i, 0)),
                  pl.BlockSpec((_BN, 1), lambda i: (i, 0)),
                  pl.BlockSpec((_BN, dh), lambda i: (i, 0))],
        out_specs=[pl.BlockSpec((_BN, dh), lambda i: (i, 0)),
                   pl.BlockSpec((_BN, 1), lambda i: (i, 0))],
        out_shape=[jax.ShapeDtypeStruct((n, dh), jnp.float32),
                   jax.ShapeDtypeStruct((n, 1), jnp.float32)],
    )(d0, d1, hw)


def _tc_layer(s0, s1, g, dinv, b, w):
    n, dh = g.shape
    m = w.shape[1]

    def body(s0_ref, s1_ref, g_ref, dinv_ref, b_ref, w_ref, o_ref):
        dv = dinv_ref[...]
        pre = dv * (s0_ref[...] + s1_ref[...] + g_ref[...]) + b_ref[...]
        h = jnp.maximum(pre, 0.0)
        o_ref[...] = dv * jnp.dot(h, w_ref[...],
                                  preferred_element_type=jnp.float32)

    return pl.pallas_call(
        body,
        grid=(n // _BN,),
        in_specs=[pl.BlockSpec((_BN, dh), lambda i: (i, 0)),
                  pl.BlockSpec((_BN, dh), lambda i: (i, 0)),
                  pl.BlockSpec((_BN, dh), lambda i: (i, 0)),
                  pl.BlockSpec((_BN, 1), lambda i: (i, 0)),
                  pl.BlockSpec((1, dh), lambda i: (0, 0)),
                  pl.BlockSpec((dh, m), lambda i: (0, 0))],
        out_specs=pl.BlockSpec((_BN, m), lambda i: (i, 0)),
        out_shape=jax.ShapeDtypeStruct((n, m), jnp.float32),
    )(s0, s1, g, dinv, b, w)


def _tc_out(s0, s1, g, dinv, b, w, bo):
    n, dh = g.shape
    m = w.shape[1]

    def body(s0_ref, s1_ref, g_ref, dinv_ref, b_ref, w_ref, bo_ref, o_ref):
        dv = dinv_ref[...]
        pre = dv * (s0_ref[...] + s1_ref[...] + g_ref[...]) + b_ref[...]
        h = jnp.maximum(pre, 0.0)
        o_ref[...] = jnp.dot(h, w_ref[...],
                             preferred_element_type=jnp.float32) + bo_ref[...]

    return pl.pallas_call(
        body,
        grid=(n // _BN,),
        in_specs=[pl.BlockSpec((_BN, dh), lambda i: (i, 0)),
                  pl.BlockSpec((_BN, dh), lambda i: (i, 0)),
                  pl.BlockSpec((_BN, dh), lambda i: (i, 0)),
                  pl.BlockSpec((_BN, 1), lambda i: (i, 0)),
                  pl.BlockSpec((1, dh), lambda i: (0, 0)),
                  pl.BlockSpec((dh, m), lambda i: (0, 0)),
                  pl.BlockSpec((1, m), lambda i: (0, 0))],
        out_specs=pl.BlockSpec((_BN, m), lambda i: (i, 0)),
        out_shape=jax.ShapeDtypeStruct((n, m), jnp.float32),
    )(s0, s1, g, dinv, b, w, bo)


def kernel(x, edge_index, edge_weight, W1, b1, W2, b2, Wc, bc):
    n, _ = x.shape
    e = edge_weight.shape[0]
    d_h = W1.shape[1]

    s_tot = -(-e // (_NS * _CHUNK))
    s_tot = ((s_tot + 5) // 6) * 6
    cpw1 = max(3, int(round(s_tot * 0.25 / 3)) * 3)
    cpw0 = s_tot - cpw1
    e_pad = _NS * s_tot * _CHUNK
    pad = e_pad - e
    nch = e_pad // _CHUNK

    src = jnp.concatenate([edge_index[0], jnp.zeros((pad,), jnp.int32)])
    dst = jnp.concatenate(
        [edge_index[1], jnp.full((pad,), n - 1, dtype=jnp.int32)])
    ew = jnp.concatenate(
        [edge_weight, jnp.full((pad,), -jnp.inf, dtype=jnp.float32)])
    packed = jnp.stack(
        [src.reshape(nch, _CHUNK), dst.reshape(nch, _CHUNK),
         lax.bitcast_convert_type(ew, jnp.int32).reshape(nch, _CHUNK)],
        axis=1).reshape(-1)

    degp = _make_deg_kernel(n, e_pad)(packed)
    hW1 = _tc_mm(x, W1)
    d0 = degp[:n].reshape(n, 1)
    d1 = degp[n:].reshape(n, 1)
    g1, dinv = _tc_finalize(d0, d1, hW1)

    agg = _make_agg_kernel(n, cpw0, cpw1, d_h)
    s1 = agg(g1, packed)
    g2 = _tc_layer(s1[:n], s1[n:], g1, dinv, b1.reshape(1, -1), W2)
    s2 = agg(g2, packed)
    out = _tc_out(s2[:n], s2[n:], g2, dinv, b2.reshape(1, -1), Wc,
                  bc.reshape(1, -1))
    return out


# speedup vs baseline: 2.7245x; 2.7245x over previous
import functools

import jax
import jax.numpy as jnp
from jax import lax
from jax.experimental import pallas as pl
from jax.experimental.pallas import tpu as pltpu
from jax.experimental.pallas import tpu_sc as plsc

_NC = 2
_NS = 16
_NW = _NC * _NS
_CHUNK = 112
_REC = 3 * _CHUNK
_BN = 2000


def _sc_mesh():
    return plsc.VectorSubcoreMesh(core_axis_name="c", subcore_axis_name="s")


def _make_deg_kernel(n, e_pad):
    cpw = e_pad // (_NW * _CHUNK)

    @functools.partial(
        pl.kernel,
        mesh=_sc_mesh(),
        out_type=jax.ShapeDtypeStruct((_NC * n,), jnp.float32),
        scratch_types=[
            pltpu.VMEM((cpw * _REC,), jnp.int32),
            pltpu.VMEM((_CHUNK,), jnp.int32),
            pltpu.VMEM((_CHUNK,), jnp.float32),
            pltpu.VMEM((n,), jnp.float32),
            pltpu.VMEM_SHARED((n,), jnp.float32),
        ],
    )
    def deg_kernel(pk_hbm, out_hbm, pk_all, dst_v, w_v, zero_v, acc):
        c = lax.axis_index("c")
        s = lax.axis_index("s")
        wid = c * _NS + s
        base = pl.multiple_of(wid * cpw * _REC, 8)

        pltpu.sync_copy(pk_hbm.at[pl.ds(base, cpw * _REC)], pk_all)

        @pl.when(s == 0)
        def _zero():
            zeros = jnp.zeros((16,), jnp.float32)

            def zb(i, carry):
                zero_v[pl.ds(pl.multiple_of(i * 16, 16), 16)] = zeros
                return carry

            lax.fori_loop(0, n // 16, zb, 0)
            pltpu.sync_copy(zero_v, acc)

        plsc.subcore_barrier()

        def body(k, carry):
            def grp(gi, carry2):
                off = pl.multiple_of(k * _REC + gi * 16, 16)
                dst_v[pl.ds(gi * 16, 16)] = pk_all[pl.ds(off + _CHUNK, 16)]
                w_v[pl.ds(gi * 16, 16)] = jnp.exp(
                    lax.bitcast_convert_type(
                        pk_all[pl.ds(off + 2 * _CHUNK, 16)], jnp.float32))
                return carry2

            lax.fori_loop(0, _CHUNK // 16, grp, 0)
            pltpu.sync_copy(w_v, acc.at[dst_v], add=True)
            return carry

        lax.fori_loop(0, cpw, body, 0)
        plsc.subcore_barrier()

        q = n // 5
        @pl.when(s < 5)
        def _write():
            pltpu.sync_copy(acc.at[pl.ds(pl.multiple_of(s * q, 8), q)],
                            zero_v.at[pl.ds(0, q)])
            pltpu.sync_copy(
                zero_v.at[pl.ds(0, q)],
                out_hbm.at[pl.ds(pl.multiple_of(c * n + s * q, 8), q)],
            )

    return deg_kernel


def _make_agg_kernel(n, cpw0, cpw1, d):
    nt0 = cpw0 // 3
    nt1 = cpw1 // 3
    slab = ((-(-n // _NS) + 7) // 8) * 8
    last_slab = n - (_NS - 1) * slab

    @functools.partial(
        pl.kernel,
        mesh=_sc_mesh(),
        out_type=jax.ShapeDtypeStruct((_NC * n, d), jnp.float32),
        scratch_types=[
            pltpu.VMEM((_CHUNK, d), jnp.float32),
            pltpu.VMEM((_CHUNK, d), jnp.float32),
            pltpu.VMEM((_CHUNK, d), jnp.float32),
            pltpu.VMEM((_REC,), jnp.int32),
            pltpu.VMEM((_REC,), jnp.int32),
            pltpu.VMEM((_REC,), jnp.int32),
            pltpu.VMEM((_CHUNK,), jnp.int32),
            pltpu.VMEM((_CHUNK,), jnp.int32),
            pltpu.VMEM((_CHUNK,), jnp.int32),
            pltpu.VMEM_SHARED((n, d), jnp.float32),
            pltpu.SemaphoreType.DMA,
            pltpu.SemaphoreType.DMA,
            pltpu.SemaphoreType.DMA,
            pltpu.SemaphoreType.DMA,
            pltpu.SemaphoreType.DMA,
            pltpu.SemaphoreType.DMA,
            pltpu.SemaphoreType.DMA,
            pltpu.SemaphoreType.DMA,
            pltpu.SemaphoreType.DMA,
        ],
    )
    def agg_kernel(g_hbm, pk_hbm, out_hbm,
                   r0, r1, r2, eb0, eb1, eb2, db0, db1, db2, acc,
                   es0, es1, es2, gs0, gs1, gs2, ss0, ss1, ss2):
        c_ax = lax.axis_index("c")
        s_ax = lax.axis_index("s")
        cbase = jnp.where(c_ax == 0, s_ax * cpw0,
                          _NS * cpw0 + s_ax * cpw1)
        nt_dyn = jnp.where(c_ax == 0, nt0, nt1)

        rows = (r0, r1, r2)
        ebs = (eb0, eb1, eb2)
        dbs = (db0, db1, db2)
        esem = (es0, es1, es2)
        gsem = (gs0, gs1, gs2)
        ssem = (ss0, ss1, ss2)

        zeros = jnp.zeros((16,), jnp.float32)

        def zb(i, carry):
            for j in range(d // 16):
                r0[i, pl.ds(j * 16, 16)] = zeros
            return carry

        lax.fori_loop(0, _CHUNK, zb, 0)

        row0 = s_ax * slab

        def zero_slab(nrows):
            full = nrows // _CHUNK
            rem = nrows - full * _CHUNK
            for i in range(full):
                pltpu.sync_copy(
                    r0,
                    acc.at[pl.ds(pl.multiple_of(row0 + i * _CHUNK, 8),
                                 _CHUNK)])
            if rem:
                pltpu.sync_copy(
                    r0.at[pl.ds(0, rem)],
                    acc.at[pl.ds(pl.multiple_of(row0 + full * _CHUNK, 8),
                                 rem)])

        @pl.when(s_ax < _NS - 1)
        def _z0():
            zero_slab(slab)

        @pl.when(s_ax == _NS - 1)
        def _z1():
            zero_slab(last_slab)

        plsc.subcore_barrier()

        def start_eload(k, u):
            gid = pl.multiple_of((cbase + k) * _REC, 8)
            pltpu.async_copy(pk_hbm.at[pl.ds(gid, _REC)], ebs[u], esem[u])

        def wait_eload(u):
            pltpu.make_async_copy(pk_hbm.at[pl.ds(0, _REC)], ebs[u],
                                  esem[u]).wait()

        def start_gather(u):
            idx = ebs[u].at[pl.ds(0, _CHUNK)]
            pltpu.async_copy(g_hbm.at[idx], rows[u], gsem[u])

        def wait_gather(u):
            pltpu.make_async_copy(g_hbm.at[pl.ds(0, _CHUNK)], rows[u],
                                  gsem[u]).wait()

        def start_scatter(u):
            pltpu.async_copy(rows[u], acc.at[dbs[u]], ssem[u], add=True)

        def wait_scatter(u):
            pltpu.make_async_copy(g_hbm.at[pl.ds(0, _CHUNK)], rows[u],
                                  ssem[u]).wait()

        def process(u):
            ru = rows[u]
            eu = ebs[u]
            du = dbs[u]

            def cp(gi, carry):
                off = pl.multiple_of(gi * 16, 16)
                du[pl.ds(off, 16)] = eu[pl.ds(off + _CHUNK, 16)]
                return carry

            lax.fori_loop(0, _CHUNK // 16, cp, 0)

            orv = eu[pl.ds(2 * _CHUNK, 16)]
            for gi in range(1, _CHUNK // 16):
                orv = orv | eu[pl.ds(2 * _CHUNK + gi * 16, 16)]

            tot = orv[0]
            for l in range(1, 16):
                tot = tot | orv[l]

            @pl.when(tot != 0)
            def _scale():
                def grp(gi, carry):
                    off = pl.multiple_of(gi * 16, 16)
                    w16 = jnp.exp(lax.bitcast_convert_type(
                        eu[pl.ds(off + 2 * _CHUNK, 16)], jnp.float32))
                    for l in range(16):
                        we = w16[l]
                        e = gi * 16 + l
                        for j in range(d // 16):
                            ru[e, pl.ds(j * 16, 16)] = (
                                ru[e, pl.ds(j * 16, 16)] * we)
                    return carry

                lax.fori_loop(0, _CHUNK // 16, grp, 0)

        start_eload(0, 0)
        start_eload(1, 1)
        wait_eload(0)
        start_gather(0)

        def body(t, carry):
            for u in range(3):
                c = t * 3 + u
                nxt = (u + 2) % 3
                prv = (u + 1) % 3
                if u == 0:
                    start_eload(c + 2, nxt)
                    wait_eload(prv)

                    @pl.when(t > 0)
                    def _ws():
                        wait_scatter(prv)
                    start_gather(prv)
                elif u == 1:
                    @pl.when(t < nt_dyn - 1)
                    def _el():
                        start_eload(c + 2, nxt)
                    wait_eload(prv)

                    @pl.when(t > 0)
                    def _ws1():
                        wait_scatter(prv)
                    start_gather(prv)
                else:
                    @pl.when(t < nt_dyn - 1)
                    def _el2():
                        start_eload(c + 2, nxt)
                    wait_scatter(prv)

                    @pl.when(t < nt_dyn - 1)
                    def _g2():
                        wait_eload(prv)
                        start_gather(prv)
                wait_gather(u)
                process(u)
                start_scatter(u)
            return carry

        lax.fori_loop(0, nt_dyn, body, 0)
        wait_scatter(1)
        wait_scatter(2)
        plsc.subcore_barrier()

        def writeout(nrows):
            pltpu.sync_copy(
                acc.at[pl.ds(pl.multiple_of(row0, 8), nrows)],
                out_hbm.at[pl.ds(pl.multiple_of(c_ax * n + row0, 8), nrows)],
            )

        @pl.when(s_ax < _NS - 1)
        def _w0():
            writeout(slab)

        @pl.when(s_ax == _NS - 1)
        def _w1():
            writeout(last_slab)

    return agg_kernel


def _tc_mm(x, w):
    n, k = x.shape
    m = w.shape[1]

    def body(x_ref, w_ref, o_ref):
        o_ref[...] = jnp.dot(x_ref[...], w_ref[...],
                             preferred_element_type=jnp.float32)

    return pl.pallas_call(
        body,
        grid=(n // _BN,),
        in_specs=[pl.BlockSpec((_BN, k), lambda i: (i, 0)),
                  pl.BlockSpec((k, m), lambda i: (0, 0))],
        out_specs=pl.BlockSpec((_BN, m), lambda i: (i, 0)),
        out_shape=jax.ShapeDtypeStruct((n, m), jnp.float32),
    )(x, w)


def _tc_finalize(d0, d1, hw):
    n, dh = hw.shape

    def body(d0_ref, d1_ref, hw_ref, g_ref, dinv_ref):
        deg = d0_ref[...] + d1_ref[...] + 1.0
        dinv = lax.rsqrt(deg)
        dinv_ref[...] = dinv
        g_ref[...] = dinv * hw_ref[...]

    return pl.pallas_call(
        body,
        grid=(n // _BN,),
        in_specs=[pl.BlockSpec((_BN, 1), lambda i: (i, 0)),
                  pl.BlockSpec((_BN, 1), lambda i: (i, 0)),
                  pl.BlockSpec((_BN, dh), lambda i: (i, 0))],
        out_specs=[pl.BlockSpec((_BN, dh), lambda i: (i, 0)),
                   pl.BlockSpec((_BN, 1), lambda i: (i, 0))],
        out_shape=[jax.ShapeDtypeStruct((n, dh), jnp.float32),
                   jax.ShapeDtypeStruct((n, 1), jnp.float32)],
    )(d0, d1, hw)


def _tc_layer(s0, s1, g, dinv, b, w):
    n, dh = g.shape
    m = w.shape[1]

    def body(s0_ref, s1_ref, g_ref, dinv_ref, b_ref, w_ref, o_ref):
        dv = dinv_ref[...]
        pre = dv * (s0_ref[...] + s1_ref[...] + g_ref[...]) + b_ref[...]
        h = jnp.maximum(pre, 0.0)
        o_ref[...] = dv * jnp.dot(h, w_ref[...],
                                  preferred_element_type=jnp.float32)

    return pl.pallas_call(
        body,
        grid=(n // _BN,),
        in_specs=[pl.BlockSpec((_BN, dh), lambda i: (i, 0)),
                  pl.BlockSpec((_BN, dh), lambda i: (i, 0)),
                  pl.BlockSpec((_BN, dh), lambda i: (i, 0)),
                  pl.BlockSpec((_BN, 1), lambda i: (i, 0)),
                  pl.BlockSpec((1, dh), lambda i: (0, 0)),
                  pl.BlockSpec((dh, m), lambda i: (0, 0))],
        out_specs=pl.BlockSpec((_BN, m), lambda i: (i, 0)),
        out_shape=jax.ShapeDtypeStruct((n, m), jnp.float32),
    )(s0, s1, g, dinv, b, w)


def _tc_out(s0, s1, g, dinv, b, w, bo):
    n, dh = g.shape
    m = w.shape[1]

    def body(s0_ref, s1_ref, g_ref, dinv_ref, b_ref, w_ref, bo_ref, o_ref):
        dv = dinv_ref[...]
        pre = dv * (s0_ref[...] + s1_ref[...] + g_ref[...]) + b_ref[...]
        h = jnp.maximum(pre, 0.0)
        o_ref[...] = jnp.dot(h, w_ref[...],
                             preferred_element_type=jnp.float32) + bo_ref[...]

    return pl.pallas_call(
        body,
        grid=(n // _BN,),
        in_specs=[pl.BlockSpec((_BN, dh), lambda i: (i, 0)),
                  pl.BlockSpec((_BN, dh), lambda i: (i, 0)),
                  pl.BlockSpec((_BN, dh), lambda i: (i, 0)),
                  pl.BlockSpec((_BN, 1), lambda i: (i, 0)),
                  pl.BlockSpec((1, dh), lambda i: (0, 0)),
                  pl.BlockSpec((dh, m), lambda i: (0, 0)),
                  pl.BlockSpec((1, m), lambda i: (0, 0))],
        out_specs=pl.BlockSpec((_BN, m), lambda i: (i, 0)),
        out_shape=jax.ShapeDtypeStruct((n, m), jnp.float32),
    )(s0, s1, g, dinv, b, w, bo)


def kernel(x, edge_index, edge_weight, W1, b1, W2, b2, Wc, bc):
    n, _ = x.shape
    e = edge_weight.shape[0]
    d_h = W1.shape[1]

    s_tot = -(-e // (_NS * _CHUNK))
    s_tot = ((s_tot + 5) // 6) * 6
    cpw1 = max(3, int(round(s_tot * 0.25 / 3)) * 3)
    cpw0 = s_tot - cpw1
    e_pad = _NS * s_tot * _CHUNK
    pad = e_pad - e
    nch = e_pad // _CHUNK

    src = jnp.concatenate([edge_index[0], jnp.zeros((pad,), jnp.int32)])
    dst = jnp.concatenate(
        [edge_index[1], jnp.full((pad,), n - 1, dtype=jnp.int32)])
    ew = jnp.concatenate(
        [edge_weight, jnp.full((pad,), -jnp.inf, dtype=jnp.float32)])
    packed = jnp.stack(
        [src.reshape(nch, _CHUNK), dst.reshape(nch, _CHUNK),
         lax.bitcast_convert_type(ew, jnp.int32).reshape(nch, _CHUNK)],
        axis=1).reshape(-1)

    degp = _make_deg_kernel(n, e_pad)(packed)
    hW1 = _tc_mm(x, W1)
    d0 = degp[:n].reshape(n, 1)
    d1 = degp[n:].reshape(n, 1)
    g1, dinv = _tc_finalize(d0, d1, hW1)

    agg = _make_agg_kernel(n, cpw0, cpw1, d_h)
    s1 = agg(g1, packed)
    g2 = _tc_layer(s1[:n], s1[n:], g1, dinv, b1.reshape(1, -1), W2)
    s2 = agg(g2, packed)
    out = _tc_out(s2[:n], s2[n:], g2, dinv, b2.reshape(1, -1), Wc,
                  bc.reshape(1, -1))
    return out
